# baseline (speedup 1.0000x reference)
import functools

import jax
import jax.numpy as jnp
from jax import lax
from jax.experimental import pallas as pl
from jax.experimental.pallas import tpu as pltpu
from jax.experimental.pallas import tpu_sc as plsc

DIM = 16


def _glove_body(i_hbm, j_hbm, wt_hbm, wtt_hbm, b_hbm, bt_hbm, out_hbm,
                iv, jv, wcol, wtcol, bv, btv, outv, sem):
    cid = lax.axis_index("c")
    sid = lax.axis_index("s")

    @pl.when(jnp.logical_and(cid == 0, sid == 0))
    def _():
        ci = pltpu.async_copy(i_hbm, iv, sem)
        cj = pltpu.async_copy(j_hbm, jv, sem)
        ci.wait()
        cj.wait()
        c3 = pltpu.async_copy(b_hbm.at[iv], bv.at[pl.ds(0, 1)], sem)
        c4 = pltpu.async_copy(bt_hbm.at[jv], btv.at[pl.ds(0, 1)], sem)
        zeros = jnp.zeros((DIM,), jnp.int32)
        ivec = plsc.load_gather(iv, [zeros])
        jvec = plsc.load_gather(jv, [zeros])
        si = ivec[0]
        sj = jvec[0]
        bi = pl.multiple_of((si // 128) * 128, 128)
        bj = pl.multiple_of((sj // 128) * 128, 128)
        c1 = pltpu.async_copy(wt_hbm.at[:, pl.ds(bi, 128)], wcol, sem)
        c2 = pltpu.async_copy(wtt_hbm.at[:, pl.ds(bj, 128)], wtcol, sem)
        c1.wait()
        c2.wait()
        c3.wait()
        c4.wait()
        rows = jnp.arange(DIM, dtype=jnp.int32)
        wi = plsc.load_gather(wcol, [rows, ivec % 128])
        wj = plsc.load_gather(wtcol, [rows, jvec % 128])
        dot = jnp.sum(wi * wj)
        r = dot + bv[...][0] + btv[...][0]
        outv[...] = jnp.full((DIM,), r, dtype=jnp.float32)
        pltpu.sync_copy(outv, out_hbm)


@jax.jit
def _glove_call(i1, j1, WT, WtT, b_flat, bt_flat):
    mesh = plsc.VectorSubcoreMesh(core_axis_name="c", subcore_axis_name="s")
    fn = functools.partial(
        pl.kernel,
        mesh=mesh,
        out_type=jax.ShapeDtypeStruct((DIM,), jnp.float32),
        scratch_types=[
            pltpu.VMEM((1,), jnp.int32),
            pltpu.VMEM((1,), jnp.int32),
            pltpu.VMEM((DIM, 128), jnp.float32),
            pltpu.VMEM((DIM, 128), jnp.float32),
            pltpu.VMEM((DIM,), jnp.float32),
            pltpu.VMEM((DIM,), jnp.float32),
            pltpu.VMEM((DIM,), jnp.float32),
            pltpu.SemaphoreType.DMA,
        ],
        compiler_params=pltpu.CompilerParams(
            needs_layout_passes=False, use_tc_tiling_on_sc=True),
    )(_glove_body)
    return fn(i1, j1, WT, WtT, b_flat, bt_flat)


def kernel(i, j, W, W_tilde, b, b_tilde):
    i1 = jnp.reshape(i, (1,)).astype(jnp.int32)
    j1 = jnp.reshape(j, (1,)).astype(jnp.int32)
    out = _glove_call(i1, j1, W.T, W_tilde.T,
                      jnp.reshape(b, (-1,)), jnp.reshape(b_tilde, (-1,)))
    return out[0]


# speedup vs baseline: 5.0688x; 5.0688x over previous
import functools

import jax
import jax.numpy as jnp
from jax import lax
from jax.experimental import pallas as pl
from jax.experimental.pallas import tpu as pltpu
from jax.experimental.pallas import tpu_sc as plsc

DIM = 16
LANE = 128


def _glove_body(i_hbm, j_hbm, wt_hbm, wtt_hbm, bt_hbm, btt_hbm, out_hbm,
                iv, jv, wblk, wtblk, bblk, btblk, outv, sem):
    cid = lax.axis_index("c")
    sid = lax.axis_index("s")

    @pl.when(jnp.logical_and(cid == 0, sid == 0))
    def _():
        ci = pltpu.async_copy(i_hbm, iv, sem)
        cj = pltpu.async_copy(j_hbm, jv, sem)
        ci.wait()
        cj.wait()
        zeros = jnp.zeros((DIM,), jnp.int32)
        ivec = plsc.load_gather(iv, [zeros])
        jvec = plsc.load_gather(jv, [zeros])
        si = ivec[0]
        sj = jvec[0]
        bi = pl.multiple_of((si // LANE) * LANE, LANE)
        bj = pl.multiple_of((sj // LANE) * LANE, LANE)
        c1 = pltpu.async_copy(wt_hbm.at[:, pl.ds(bi, LANE)], wblk, sem)
        c2 = pltpu.async_copy(wtt_hbm.at[:, pl.ds(bj, LANE)], wtblk, sem)
        c3 = pltpu.async_copy(bt_hbm.at[:, pl.ds(bi, LANE)], bblk, sem)
        c4 = pltpu.async_copy(btt_hbm.at[:, pl.ds(bj, LANE)], btblk, sem)
        c1.wait()
        c2.wait()
        c3.wait()
        c4.wait()
        rows = jnp.arange(DIM, dtype=jnp.int32)
        ci16 = ivec % LANE
        cj16 = jvec % LANE
        wi = plsc.load_gather(wblk, [rows, ci16])
        wj = plsc.load_gather(wtblk, [rows, cj16])
        bval = plsc.load_gather(bblk, [zeros, ci16])
        btval = plsc.load_gather(btblk, [zeros, cj16])
        dot = jnp.sum(wi * wj)
        r = dot + bval[0] + btval[0]
        outv[...] = jnp.full((DIM,), r, dtype=jnp.float32)
        pltpu.sync_copy(outv, out_hbm)


@jax.jit
def _glove_call(i1, j1, WT, WtT, bT, btT):
    mesh = plsc.VectorSubcoreMesh(core_axis_name="c", subcore_axis_name="s")
    fn = functools.partial(
        pl.kernel,
        mesh=mesh,
        out_type=jax.ShapeDtypeStruct((DIM,), jnp.float32),
        scratch_types=[
            pltpu.VMEM((1,), jnp.int32),
            pltpu.VMEM((1,), jnp.int32),
            pltpu.VMEM((DIM, LANE), jnp.float32),
            pltpu.VMEM((DIM, LANE), jnp.float32),
            pltpu.VMEM((1, LANE), jnp.float32),
            pltpu.VMEM((1, LANE), jnp.float32),
            pltpu.VMEM((DIM,), jnp.float32),
            pltpu.SemaphoreType.DMA,
        ],
        compiler_params=pltpu.CompilerParams(
            needs_layout_passes=False, use_tc_tiling_on_sc=True),
    )(_glove_body)
    return fn(i1, j1, WT, WtT, bT, btT)


def kernel(i, j, W, W_tilde, b, b_tilde):
    i1 = jnp.reshape(i, (1,)).astype(jnp.int32)
    j1 = jnp.reshape(j, (1,)).astype(jnp.int32)
    out = _glove_call(i1, j1, W.T, W_tilde.T, b.T, b_tilde.T)
    return out[0]


# speedup vs baseline: 5.5286x; 1.0907x over previous
import functools

import jax
import jax.numpy as jnp
from jax import lax
from jax.experimental import pallas as pl
from jax.experimental.pallas import tpu as pltpu
from jax.experimental.pallas import tpu_sc as plsc

DIM = 16
LANE = 128


def _glove_body(ij_hbm, wt_hbm, wtt_hbm, bt_hbm, btt_hbm, out_hbm,
                ijv, wblk, wtblk, bblk, btblk, outv, sem):
    sid = lax.axis_index("s")

    @pl.when(sid == 0)
    def _():
        pltpu.sync_copy(ij_hbm, ijv)
        zeros = jnp.zeros((DIM,), jnp.int32)
        ivec = plsc.load_gather(ijv, [zeros])
        jvec = plsc.load_gather(ijv, [zeros + 1])
        si = ivec[0]
        sj = jvec[0]
        bi = pl.multiple_of((si // LANE) * LANE, LANE)
        bj = pl.multiple_of((sj // LANE) * LANE, LANE)
        c1 = pltpu.async_copy(wt_hbm.at[:, pl.ds(bi, LANE)], wblk, sem)
        c2 = pltpu.async_copy(wtt_hbm.at[:, pl.ds(bj, LANE)], wtblk, sem)
        c3 = pltpu.async_copy(bt_hbm.at[:, pl.ds(bi, LANE)], bblk, sem)
        c4 = pltpu.async_copy(btt_hbm.at[:, pl.ds(bj, LANE)], btblk, sem)
        c1.wait()
        c2.wait()
        c3.wait()
        c4.wait()
        rows = jnp.arange(DIM, dtype=jnp.int32)
        ci16 = ivec % LANE
        cj16 = jvec % LANE
        wi = plsc.load_gather(wblk, [rows, ci16])
        wj = plsc.load_gather(wtblk, [rows, cj16])
        bval = plsc.load_gather(bblk, [zeros, ci16])
        btval = plsc.load_gather(btblk, [zeros, cj16])
        dot = jnp.sum(wi * wj)
        r = dot + bval[0] + btval[0]
        outv[...] = jnp.full((DIM,), r, dtype=jnp.float32)
        pltpu.sync_copy(outv, out_hbm)


@jax.jit
def _glove_call(ij, WT, WtT, bT, btT):
    mesh = plsc.VectorSubcoreMesh(
        core_axis_name="c", subcore_axis_name="s", num_cores=1)
    fn = functools.partial(
        pl.kernel,
        mesh=mesh,
        out_type=jax.ShapeDtypeStruct((DIM,), jnp.float32),
        scratch_types=[
            pltpu.VMEM((2,), jnp.int32),
            pltpu.VMEM((DIM, LANE), jnp.float32),
            pltpu.VMEM((DIM, LANE), jnp.float32),
            pltpu.VMEM((1, LANE), jnp.float32),
            pltpu.VMEM((1, LANE), jnp.float32),
            pltpu.VMEM((DIM,), jnp.float32),
            pltpu.SemaphoreType.DMA,
        ],
        compiler_params=pltpu.CompilerParams(
            needs_layout_passes=False, use_tc_tiling_on_sc=True),
    )(_glove_body)
    return fn(ij, WT, WtT, bT, btT)


def kernel(i, j, W, W_tilde, b, b_tilde):
    ij = jnp.stack([i, j]).astype(jnp.int32)
    out = _glove_call(ij, W.T, W_tilde.T, b.T, b_tilde.T)
    return out[0]
